# Initial kernel scaffold; baseline (speedup 1.0000x reference)
import jax
import jax.numpy as jnp
from jax.experimental import pallas as pl


def kernel(cls_p3, cls_p4, cls_p5, cls_p6, cls_p7, cnt_p3, cnt_p4, cnt_p5, cnt_p6, cnt_p7, reg_p3, reg_p4, reg_p5, reg_p6, reg_p7):
    raise NotImplementedError("write your pallas kernel here")


# speedup vs baseline: 18.4571x; 18.4571x over previous
import jax
import jax.numpy as jnp
import numpy as np
from jax import lax
from jax.experimental import pallas as pl
from jax.experimental.pallas import tpu as pltpu

_STRIDES = (8, 16, 32, 64, 128)
_LEVELS = ((64, 64), (32, 32), (16, 16), (8, 8), (4, 4))
_B = 8
_NC = 80
_N = 5456
_NPAD = 5504
_K = 1000
_KPAD = 1024
_SCORE_THR = 0.05
_NMS_IOU = 0.6


def _make_coords():
    cx = np.full((1, _NPAD), 0.0, np.float32)
    cy = np.full((1, _NPAD), 0.0, np.float32)
    off = 0
    for (h, w), s in zip(_LEVELS, _STRIDES):
        sx = np.arange(w, dtype=np.float32) * s + s // 2
        sy = np.arange(h, dtype=np.float32) * s + s // 2
        ys, xs = np.meshgrid(sy, sx, indexing="ij")
        cx[0, off:off + h * w] = xs.reshape(-1)
        cy[0, off:off + h * w] = ys.reshape(-1)
        off += h * w
    return cx, cy

_CX, _CY = _make_coords()
_LVL_OFF = (0, 4096, 5120, 5376, 5440)


def _score_decode_body(*refs):
    cls_refs = refs[0:5]
    cnt_refs = refs[5:10]
    reg_refs = refs[10:15]
    cx_ref, cy_ref = refs[15], refs[16]
    s_ref, c_ref, x1_ref, y1_ref, x2_ref, y2_ref, t_ref = refs[17:24]

    per_level = []
    for lvl in range(5):
        hw = _LEVELS[lvl][0] * _LEVELS[lvl][1]
        off = _LVL_OFF[lvl]
        logits = cls_refs[lvl][...]
        sig = jax.nn.sigmoid(logits)
        m = jnp.max(sig, axis=1)
        cidx = lax.broadcasted_iota(jnp.int32, (_B, _NC, hw), 1)
        am = jnp.min(jnp.where(sig == m[:, None, :], cidx, 10**9), axis=1)
        cls_out = (am + 1).astype(jnp.float32)
        cnt_sig = jax.nn.sigmoid(cnt_refs[lvl][...])[:, 0, :]
        score = jnp.sqrt(m * cnt_sig)
        reg = reg_refs[lvl][...]
        cx = cx_ref[:, off:off + hw]
        cy = cy_ref[:, off:off + hw]
        x1 = cx - reg[:, 0, :]
        y1 = cy - reg[:, 1, :]
        x2 = cx + reg[:, 2, :]
        y2 = cy + reg[:, 3, :]
        per_level.append((score, cls_out, x1, y1, x2, y2))

    outs = (s_ref, c_ref, x1_ref, y1_ref, x2_ref, y2_ref)
    pads = (-1.0, 0.0, 0.0, 0.0, 0.0, 0.0)
    for lvl in range(3):
        hw = _LEVELS[lvl][0] * _LEVELS[lvl][1]
        off = _LVL_OFF[lvl]
        for ref, val in zip(outs, per_level[lvl]):
            ref[:, off:off + hw] = val
    for j, (ref, padv) in enumerate(zip(outs, pads)):
        tile = jnp.concatenate(
            [per_level[3][j], per_level[4][j],
             jnp.full((_B, 48), padv, jnp.float32)], axis=1)
        ref[:, _LVL_OFF[3]:_LVL_OFF[3] + 128] = tile

    sb = lax.bitcast_convert_type(s_ref[...], jnp.int32)
    lane = lax.broadcasted_iota(jnp.int32, (_B, _NPAD), 1)
    real = lane < _N

    def bs_body(_, carry):
        lo, hi = carry
        mid = lo + lax.shift_right_logical(hi - lo, 1)
        cnt = jnp.sum(jnp.where((sb >= mid) & real, 1, 0), axis=1,
                      keepdims=True)
        ge = cnt >= _K
        return jnp.where(ge, mid, lo), jnp.where(ge, hi, mid)

    lo0 = jnp.zeros((_B, 1), jnp.int32)
    hi0 = jnp.full((_B, 1), 0x7F800000, jnp.int32)
    lo, hi = lax.fori_loop(0, 31, bs_body, (lo0, hi0))
    t_ref[...] = lo


def _score_decode(cls_list, cnt_list, reg_list):
    out_sh = [jax.ShapeDtypeStruct((_B, _NPAD), jnp.float32)] * 6 + \
             [jax.ShapeDtypeStruct((_B, 1), jnp.int32)]
    fn = pl.pallas_call(_score_decode_body, out_shape=out_sh)
    return fn(*cls_list, *cnt_list, *reg_list, jnp.asarray(_CX), jnp.asarray(_CY))


def _prefix_sum_lanes(x, lane):
    d = 1
    while d < _NPAD:
        x = x + jnp.where(lane >= d, pltpu.roll(x, d, 1), 0.0)
        d *= 2
    return x


def _compact_body(s_ref, c_ref, x1_ref, y1_ref, x2_ref, y2_ref, t_ref,
                  so_ref, co_ref, xo1_ref, yo1_ref, xo2_ref, yo2_ref):
    s = s_ref[...]
    sb = lax.bitcast_convert_type(s, jnp.int32)
    t = t_ref[...]
    lane = lax.broadcasted_iota(jnp.int32, (_B, _NPAD), 1)
    real = lane < _N
    sel_gt = (sb > t) & real
    sel_eq = (sb == t) & real

    psg = _prefix_sum_lanes(sel_gt.astype(jnp.float32), lane)
    pse = _prefix_sum_lanes(sel_eq.astype(jnp.float32), lane)
    gtot = jnp.max(psg, axis=1, keepdims=True)
    pos = jnp.where(sel_gt, psg - 1.0,
                    jnp.where(sel_eq, gtot + (pse - 1.0), -1.0))

    posi = pos.astype(jnp.int32)
    chans = [s, c_ref[...], x1_ref[...], y1_ref[...], x2_ref[...], y2_ref[...]]
    orefs = (so_ref, co_ref, xo1_ref, yo1_ref, xo2_ref, yo2_ref)
    r_iota = lax.broadcasted_iota(jnp.int32, (_KPAD, _NPAD), 0)
    for b in range(_B):
        posb = posi[b:b + 1, :]
        p2 = jnp.where(posb == r_iota, 1.0, 0.0).astype(jnp.bfloat16)
        vb = jnp.concatenate([ch[b:b + 1, :] for ch in chans], axis=0)
        hi = vb.astype(jnp.bfloat16)
        r1 = vb - hi.astype(jnp.float32)
        md = r1.astype(jnp.bfloat16)
        lo2 = (r1 - md.astype(jnp.float32)).astype(jnp.bfloat16)
        dot = lambda a: lax.dot_general(
            a, p2, (((1,), (1,)), ((), ())),
            preferred_element_type=jnp.float32)
        acc = dot(hi) + dot(md) + dot(lo2)
        for ch in range(6):
            orefs[ch][b:b + 1, :] = acc[ch:ch + 1, :]


def _compact(s, c, x1, y1, x2, y2, t):
    out_sh = [jax.ShapeDtypeStruct((_B, _KPAD), jnp.float32)] * 6
    fn = pl.pallas_call(_compact_body, out_shape=out_sh)
    return fn(s, c, x1, y1, x2, y2, t)


def _nms_body(sc_ref, cl_ref, x1_ref, y1_ref, x2_ref, y2_ref,
              so_ref, co_ref, bx1_ref, by1_ref, bx2_ref, by2_ref,
              x1s, y1s, x2s, y2s, ar, kp, rkT, rk, cacc):
    scores = sc_ref[...]
    classes = cl_ref[...]
    x1, y1, x2, y2 = x1_ref[...], y1_ref[...], x2_ref[...], y2_ref[...]

    sb = lax.bitcast_convert_type(scores, jnp.int32)
    sbT = jnp.transpose(sb)
    qi = lax.broadcasted_iota(jnp.int32, (_KPAD, _KPAD), 1)
    pi = lax.broadcasted_iota(jnp.int32, (_KPAD, _KPAD), 0)
    for b in range(_B):
        colb = sbT[:, b:b + 1]
        rowb = sb[b:b + 1, :]
        before = (rowb > colb) | ((rowb == colb) & (qi < pi))
        rkT[:, b:b + 1] = jnp.sum(before.astype(jnp.float32), axis=1,
                                  keepdims=True)
    rk[...] = jnp.transpose(rkT[...])

    lane = lax.broadcasted_iota(jnp.int32, (_B, _KPAD), 1)
    live = rk[...] < float(_K)
    neg = jnp.float32(-3.0e38)
    mc = jnp.max(
        jnp.maximum(jnp.maximum(jnp.where(live, x1, neg), jnp.where(live, y1, neg)),
                    jnp.maximum(jnp.where(live, x2, neg), jnp.where(live, y2, neg))),
        axis=1, keepdims=True)
    mcp1 = mc + 1.0
    offs = classes * mcp1
    x1s[...] = x1 + offs
    y1s[...] = y1 + offs
    x2s[...] = x2 + offs
    y2s[...] = y2 + offs
    ar[...] = (x2s[...] - x1s[...] + 1.0) * (y2s[...] - y1s[...] + 1.0)
    kp[...] = (scores >= _SCORE_THR).astype(jnp.float32)

    so_ref[...] = jnp.zeros((_B, _KPAD), jnp.float32)
    cacc[...] = jnp.zeros((_B, _KPAD), jnp.float32)
    bx1_ref[...] = jnp.zeros((_B, _KPAD), jnp.float32)
    by1_ref[...] = jnp.zeros((_B, _KPAD), jnp.float32)
    bx2_ref[...] = jnp.zeros((_B, _KPAD), jnp.float32)
    by2_ref[...] = jnp.zeros((_B, _KPAD), jnp.float32)

    def body(k, _):
        kf = k.astype(jnp.float32)
        ohr = jnp.where(rk[...] == kf, 1.0, 0.0)
        ext = lambda a: jnp.sum(a * ohr, axis=1, keepdims=True)
        s_k = ext(scores)
        c_k = ext(classes)
        rx1k, ry1k, rx2k, ry2k = ext(x1), ext(y1), ext(x2), ext(y2)
        kpk = ext(kp[...])
        offk = c_k * mcp1
        x1k = rx1k + offk
        y1k = ry1k + offk
        x2k = rx2k + offk
        y2k = ry2k + offk
        ark = (x2k - x1k + 1.0) * (y2k - y1k + 1.0)
        xx1 = jnp.maximum(x1k, x1s[...])
        yy1 = jnp.maximum(y1k, y1s[...])
        xx2 = jnp.minimum(x2k, x2s[...])
        yy2 = jnp.minimum(y2k, y2s[...])
        inter = jnp.maximum(xx2 - xx1, 0.0) * jnp.maximum(yy2 - yy1, 0.0)
        iou = inter / (ark + ar[...] - inter)
        sup = (iou > _NMS_IOU) & (rk[...] > kf) & (kpk > 0.0)
        kp[...] = jnp.where(sup, 0.0, kp[...])
        ohl = jnp.where(lane == k, kpk, 0.0)
        so_ref[...] += s_k * ohl
        cacc[...] += c_k * ohl
        bx1_ref[...] += rx1k * ohl
        by1_ref[...] += ry1k * ohl
        bx2_ref[...] += rx2k * ohl
        by2_ref[...] += ry2k * ohl
        return 0

    lax.fori_loop(0, _K, body, 0)
    co_ref[...] = cacc[...].astype(jnp.int32)


def _nms(topv, topc, tx1, ty1, tx2, ty2):
    out_sh = [jax.ShapeDtypeStruct((_B, _KPAD), jnp.float32),
              jax.ShapeDtypeStruct((_B, _KPAD), jnp.int32)] + \
             [jax.ShapeDtypeStruct((_B, _KPAD), jnp.float32)] * 4
    fn = pl.pallas_call(
        _nms_body,
        out_shape=out_sh,
        scratch_shapes=[pltpu.VMEM((_B, _KPAD), jnp.float32)] * 6
        + [pltpu.VMEM((_KPAD, _B), jnp.float32),
           pltpu.VMEM((_B, _KPAD), jnp.float32),
           pltpu.VMEM((_B, _KPAD), jnp.float32)],
    )
    return fn(topv, topc, tx1, ty1, tx2, ty2)


def kernel(cls_p3, cls_p4, cls_p5, cls_p6, cls_p7,
           cnt_p3, cnt_p4, cnt_p5, cnt_p6, cnt_p7,
           reg_p3, reg_p4, reg_p5, reg_p6, reg_p7):
    cls_list = [c.reshape(_B, _NC, -1)
                for c in (cls_p3, cls_p4, cls_p5, cls_p6, cls_p7)]
    cnt_list = [c.reshape(_B, 1, -1)
                for c in (cnt_p3, cnt_p4, cnt_p5, cnt_p6, cnt_p7)]
    reg_list = [r.reshape(_B, 4, -1)
                for r in (reg_p3, reg_p4, reg_p5, reg_p6, reg_p7)]

    scores, classes, x1, y1, x2, y2, t = _score_decode(cls_list, cnt_list,
                                                       reg_list)
    cs, cc, cx1, cy1, cx2, cy2 = _compact(scores, classes, x1, y1, x2, y2, t)
    so, co, bx1, by1, bx2, by2 = _nms(cs, cc, cx1, cy1, cx2, cy2)

    boxes = jnp.stack([bx1[:, :_K], by1[:, :_K], bx2[:, :_K], by2[:, :_K]],
                      axis=-1)
    return so[:, :_K], co[:, :_K], boxes


# speedup vs baseline: 20.5953x; 1.1159x over previous
import jax
import jax.numpy as jnp
import numpy as np
from jax import lax
from jax.experimental import pallas as pl
from jax.experimental.pallas import tpu as pltpu
from jax.experimental.pallas import tpu_sc as plsc

_STRIDES = (8, 16, 32, 64, 128)
_LEVELS = ((64, 64), (32, 32), (16, 16), (8, 8), (4, 4))
_B = 8
_NC = 80
_N = 5456
_NPAD = 5504
_K = 1000
_KPAD = 1024
_SCORE_THR = 0.05
_NMS_IOU = 0.6


def _make_coords():
    cx = np.full((1, _NPAD), 0.0, np.float32)
    cy = np.full((1, _NPAD), 0.0, np.float32)
    off = 0
    for (h, w), s in zip(_LEVELS, _STRIDES):
        sx = np.arange(w, dtype=np.float32) * s + s // 2
        sy = np.arange(h, dtype=np.float32) * s + s // 2
        ys, xs = np.meshgrid(sy, sx, indexing="ij")
        cx[0, off:off + h * w] = xs.reshape(-1)
        cy[0, off:off + h * w] = ys.reshape(-1)
        off += h * w
    return cx, cy

_CX, _CY = _make_coords()
_LVL_OFF = (0, 4096, 5120, 5376, 5440)


def _score_decode_body(*refs):
    cls_refs = refs[0:5]
    cnt_refs = refs[5:10]
    reg_refs = refs[10:15]
    cx_ref, cy_ref = refs[15], refs[16]
    s_ref, c_ref, x1_ref, y1_ref, x2_ref, y2_ref, t_ref = refs[17:24]

    per_level = []
    for lvl in range(5):
        hw = _LEVELS[lvl][0] * _LEVELS[lvl][1]
        off = _LVL_OFF[lvl]
        logits = cls_refs[lvl][...]
        sig = jax.nn.sigmoid(logits)
        m = jnp.max(sig, axis=1)
        cidx = lax.broadcasted_iota(jnp.int32, (_B, _NC, hw), 1)
        am = jnp.min(jnp.where(sig == m[:, None, :], cidx, 10**9), axis=1)
        cls_out = (am + 1).astype(jnp.float32)
        cnt_sig = jax.nn.sigmoid(cnt_refs[lvl][...])[:, 0, :]
        score = jnp.sqrt(m * cnt_sig)
        reg = reg_refs[lvl][...]
        cx = cx_ref[:, off:off + hw]
        cy = cy_ref[:, off:off + hw]
        x1 = cx - reg[:, 0, :]
        y1 = cy - reg[:, 1, :]
        x2 = cx + reg[:, 2, :]
        y2 = cy + reg[:, 3, :]
        per_level.append((score, cls_out, x1, y1, x2, y2))

    outs = (s_ref, c_ref, x1_ref, y1_ref, x2_ref, y2_ref)
    pads = (-1.0, 0.0, 0.0, 0.0, 0.0, 0.0)
    for lvl in range(3):
        hw = _LEVELS[lvl][0] * _LEVELS[lvl][1]
        off = _LVL_OFF[lvl]
        for ref, val in zip(outs, per_level[lvl]):
            ref[:, off:off + hw] = val
    for j, (ref, padv) in enumerate(zip(outs, pads)):
        tile = jnp.concatenate(
            [per_level[3][j], per_level[4][j],
             jnp.full((_B, 48), padv, jnp.float32)], axis=1)
        ref[:, _LVL_OFF[3]:_LVL_OFF[3] + 128] = tile

    sb = lax.bitcast_convert_type(s_ref[...], jnp.int32)
    lane = lax.broadcasted_iota(jnp.int32, (_B, _NPAD), 1)
    real = lane < _N

    def bs_body(_, carry):
        lo, hi = carry
        mid = lo + lax.shift_right_logical(hi - lo, 1)
        cnt = jnp.sum(jnp.where((sb >= mid) & real, 1, 0), axis=1,
                      keepdims=True)
        ge = cnt >= _K
        return jnp.where(ge, mid, lo), jnp.where(ge, hi, mid)

    lo0 = jnp.zeros((_B, 1), jnp.int32)
    hi0 = jnp.full((_B, 1), 0x7F800000, jnp.int32)
    lo, hi = lax.fori_loop(0, 31, bs_body, (lo0, hi0))
    t_ref[...] = jnp.broadcast_to(lo, (_B, 16))


def _score_decode(cls_list, cnt_list, reg_list):
    out_sh = [jax.ShapeDtypeStruct((_B, _NPAD), jnp.float32)] * 6 + \
             [jax.ShapeDtypeStruct((_B, 16), jnp.int32)]
    fn = pl.pallas_call(_score_decode_body, out_shape=out_sh)
    return fn(*cls_list, *cnt_list, *reg_list, jnp.asarray(_CX), jnp.asarray(_CY))


def _prefix_sum_lanes(x, lane):
    d = 1
    while d < _NPAD:
        x = x + jnp.where(lane >= d, pltpu.roll(x, d, 1), 0.0)
        d *= 2
    return x


def _compact_body(s_ref, c_ref, x1_ref, y1_ref, x2_ref, y2_ref, t_ref,
                  so_ref, co_ref, xo1_ref, yo1_ref, xo2_ref, yo2_ref):
    s = s_ref[...]
    sb = lax.bitcast_convert_type(s, jnp.int32)
    t = t_ref[:, 0:1]
    lane = lax.broadcasted_iota(jnp.int32, (_B, _NPAD), 1)
    real = lane < _N
    sel_gt = (sb > t) & real
    sel_eq = (sb == t) & real

    psg = _prefix_sum_lanes(sel_gt.astype(jnp.float32), lane)
    pse = _prefix_sum_lanes(sel_eq.astype(jnp.float32), lane)
    gtot = jnp.max(psg, axis=1, keepdims=True)
    pos = jnp.where(sel_gt, psg - 1.0,
                    jnp.where(sel_eq, gtot + (pse - 1.0), -1.0))

    posi = pos.astype(jnp.int32)
    chans = [s, c_ref[...], x1_ref[...], y1_ref[...], x2_ref[...], y2_ref[...]]
    orefs = (so_ref, co_ref, xo1_ref, yo1_ref, xo2_ref, yo2_ref)
    r_iota = lax.broadcasted_iota(jnp.int32, (_KPAD, _NPAD), 0)
    for b in range(_B):
        posb = posi[b:b + 1, :]
        p2 = jnp.where(posb == r_iota, 1.0, 0.0).astype(jnp.bfloat16)
        vb = jnp.concatenate([ch[b:b + 1, :] for ch in chans], axis=0)
        hi = vb.astype(jnp.bfloat16)
        r1 = vb - hi.astype(jnp.float32)
        md = r1.astype(jnp.bfloat16)
        lo2 = (r1 - md.astype(jnp.float32)).astype(jnp.bfloat16)
        dot = lambda a: lax.dot_general(
            a, p2, (((1,), (1,)), ((), ())),
            preferred_element_type=jnp.float32)
        acc = dot(hi) + dot(md) + dot(lo2)
        for ch in range(6):
            orefs[ch][b:b + 1, :] = acc[ch:ch + 1, :]


def _compact(s, c, x1, y1, x2, y2, t):
    out_sh = [jax.ShapeDtypeStruct((_B, _KPAD), jnp.float32)] * 6
    fn = pl.pallas_call(_compact_body, out_shape=out_sh)
    return fn(s, c, x1, y1, x2, y2, t)


_T = 128
_NBLK = _KPAD // _T


def _nms_body(sc_ref, cl_ref, x1_ref, y1_ref, x2_ref, y2_ref,
              so_ref, co_ref, bx1_ref, by1_ref, bx2_ref, by2_ref,
              x1s, y1s, x2s, y2s, ar, kp, x1sT, y1sT, x2sT, y2sT, arT, kpT):
    sb = lax.bitcast_convert_type(sc_ref[...], jnp.int32)
    sbT = jnp.transpose(sb)
    qi = lax.broadcasted_iota(jnp.int32, (_KPAD, _KPAD), 1)
    pi = lax.broadcasted_iota(jnp.int32, (_KPAD, _KPAD), 0)
    r_row = lax.broadcasted_iota(jnp.int32, (_KPAD, _KPAD), 1)
    chans = [sc_ref[...], cl_ref[...], x1_ref[...], y1_ref[...],
             x2_ref[...], y2_ref[...]]
    sorted_chans = []
    for b in range(_B):
        colb = sbT[:, b:b + 1]
        rowb = sb[b:b + 1, :]
        before = (rowb > colb) | ((rowb == colb) & (qi < pi))
        rkp = jnp.sum(before.astype(jnp.float32), axis=1,
                      keepdims=True).astype(jnp.int32)
        oh = jnp.where(rkp == r_row, 1.0, 0.0).astype(jnp.bfloat16)
        vb = jnp.concatenate([ch[b:b + 1, :] for ch in chans], axis=0)
        hi = vb.astype(jnp.bfloat16)
        r1 = vb - hi.astype(jnp.float32)
        md = r1.astype(jnp.bfloat16)
        lo2 = (r1 - md.astype(jnp.float32)).astype(jnp.bfloat16)
        dot = lambda a: lax.dot_general(
            a, oh, (((1,), (0,)), ((), ())),
            preferred_element_type=jnp.float32)
        sorted_chans.append(dot(hi) + dot(md) + dot(lo2))

    scores = jnp.concatenate([sc[0:1, :] for sc in sorted_chans], axis=0)
    classes = jnp.concatenate([sc[1:2, :] for sc in sorted_chans], axis=0)
    x1 = jnp.concatenate([sc[2:3, :] for sc in sorted_chans], axis=0)
    y1 = jnp.concatenate([sc[3:4, :] for sc in sorted_chans], axis=0)
    x2 = jnp.concatenate([sc[4:5, :] for sc in sorted_chans], axis=0)
    y2 = jnp.concatenate([sc[5:6, :] for sc in sorted_chans], axis=0)

    lane = lax.broadcasted_iota(jnp.int32, (_B, _KPAD), 1)
    live = lane < _K
    neg = jnp.float32(-3.0e38)
    mc = jnp.max(
        jnp.maximum(jnp.maximum(jnp.where(live, x1, neg), jnp.where(live, y1, neg)),
                    jnp.maximum(jnp.where(live, x2, neg), jnp.where(live, y2, neg))),
        axis=1, keepdims=True)
    mcp1 = mc + 1.0
    offs = classes * mcp1
    x1s[...] = x1 + offs
    y1s[...] = y1 + offs
    x2s[...] = x2 + offs
    y2s[...] = y2 + offs
    ar[...] = (x2s[...] - x1s[...] + 1.0) * (y2s[...] - y1s[...] + 1.0)
    kp[...] = (scores >= _SCORE_THR).astype(jnp.float32)
    x1sT[...] = jnp.transpose(x1s[...])
    y1sT[...] = jnp.transpose(y1s[...])
    x2sT[...] = jnp.transpose(x2s[...])
    y2sT[...] = jnp.transpose(y2s[...])
    arT[...] = jnp.transpose(ar[...])

    lane_t = lax.broadcasted_iota(jnp.int32, (_B, _T), 1)
    for blk in range(_NBLK):
        base = blk * _T
        nb = min(_T, _K - base)
        bx1 = x1s[:, base:base + _T]
        by1 = y1s[:, base:base + _T]
        bx2 = x2s[:, base:base + _T]
        by2 = y2s[:, base:base + _T]
        bar = ar[:, base:base + _T]

        def body(i, kb):
            oh = jnp.where(lane_t == i, 1.0, 0.0)
            ext = lambda a: jnp.sum(a * oh, axis=1, keepdims=True)
            cx1, cy1, cx2, cy2, car = map(ext, (bx1, by1, bx2, by2, bar))
            ck = jnp.sum(kb * oh, axis=1, keepdims=True)
            xx1 = jnp.maximum(cx1, bx1)
            yy1 = jnp.maximum(cy1, by1)
            xx2 = jnp.minimum(cx2, bx2)
            yy2 = jnp.minimum(cy2, by2)
            inter = jnp.maximum(xx2 - xx1, 0.0) * jnp.maximum(yy2 - yy1, 0.0)
            iou = inter / (car + bar - inter)
            sup = (iou > _NMS_IOU) & (lane_t > i) & (ck > 0.0)
            return jnp.where(sup, 0.0, kb)

        kb = lax.fori_loop(0, nb, body, kp[:, base:base + _T])
        kp[:, base:base + _T] = kb

        if blk < _NBLK - 1:
            rest = _KPAD - base - _T
            kbT = jnp.transpose(kb)
            for b in range(_B):
                cols = lambda ref: ref[base:base + _T, b:b + 1]
                rx1 = x1s[b:b + 1, base + _T:]
                ry1 = y1s[b:b + 1, base + _T:]
                rx2 = x2s[b:b + 1, base + _T:]
                ry2 = y2s[b:b + 1, base + _T:]
                rar = ar[b:b + 1, base + _T:]
                xx1 = jnp.maximum(cols(x1sT), rx1)
                yy1 = jnp.maximum(cols(y1sT), ry1)
                xx2 = jnp.minimum(cols(x2sT), rx2)
                yy2 = jnp.minimum(cols(y2sT), ry2)
                inter = jnp.maximum(xx2 - xx1, 0.0) * jnp.maximum(yy2 - yy1, 0.0)
                iou = inter / (cols(arT) + rar - inter)
                supm = (iou > _NMS_IOU) & (kbT[:, b:b + 1] > 0.0)
                supj = jnp.max(supm.astype(jnp.float32), axis=0, keepdims=True)
                kp[b:b + 1, base + _T:] = jnp.where(
                    supj > 0.0, 0.0, kp[b:b + 1, base + _T:])

    kf = kp[...]
    so_ref[...] = scores * kf
    co_ref[...] = (classes * kf).astype(jnp.int32)
    bx1_ref[...] = x1 * kf
    by1_ref[...] = y1 * kf
    bx2_ref[...] = x2 * kf
    by2_ref[...] = y2 * kf


def _nms(topv, topc, tx1, ty1, tx2, ty2):
    out_sh = [jax.ShapeDtypeStruct((_B, _KPAD), jnp.float32),
              jax.ShapeDtypeStruct((_B, _KPAD), jnp.int32)] + \
             [jax.ShapeDtypeStruct((_B, _KPAD), jnp.float32)] * 4
    fn = pl.pallas_call(
        _nms_body,
        out_shape=out_sh,
        scratch_shapes=[pltpu.VMEM((_B, _KPAD), jnp.float32)] * 6
        + [pltpu.VMEM((_KPAD, _B), jnp.float32)] * 6,
    )
    return fn(topv, topc, tx1, ty1, tx2, ty2)


def kernel(cls_p3, cls_p4, cls_p5, cls_p6, cls_p7,
           cnt_p3, cnt_p4, cnt_p5, cnt_p6, cnt_p7,
           reg_p3, reg_p4, reg_p5, reg_p6, reg_p7):
    cls_list = [c.reshape(_B, _NC, -1)
                for c in (cls_p3, cls_p4, cls_p5, cls_p6, cls_p7)]
    cnt_list = [c.reshape(_B, 1, -1)
                for c in (cnt_p3, cnt_p4, cnt_p5, cnt_p6, cnt_p7)]
    reg_list = [r.reshape(_B, 4, -1)
                for r in (reg_p3, reg_p4, reg_p5, reg_p6, reg_p7)]

    scores, classes, x1, y1, x2, y2, t = _score_decode(cls_list, cnt_list,
                                                       reg_list)
    cs, cc, cx1, cy1, cx2, cy2 = _compact(scores, classes, x1, y1, x2, y2, t)
    so, co, bx1, by1, bx2, by2 = _nms(cs, cc, cx1, cy1, cx2, cy2)

    boxes = jnp.stack([bx1[:, :_K], by1[:, :_K], bx2[:, :_K], by2[:, :_K]],
                      axis=-1)
    return so[:, :_K], co[:, :_K], boxes


# speedup vs baseline: 24.6275x; 1.1958x over previous
import dataclasses

import jax
import jax.numpy as jnp
import numpy as np
from jax import lax
from jax.experimental import pallas as pl
from jax.experimental.pallas import tpu as pltpu
from jax.experimental.pallas import tpu_sc as plsc

_STRIDES = (8, 16, 32, 64, 128)
_LEVELS = ((64, 64), (32, 32), (16, 16), (8, 8), (4, 4))
_B = 8
_NC = 80
_N = 5456
_NPAD = 5504
_K = 1000
_KPAD = 1024
_SCORE_THR = 0.05
_NMS_IOU = 0.6


def _make_coords():
    cx = np.full((1, _NPAD), 0.0, np.float32)
    cy = np.full((1, _NPAD), 0.0, np.float32)
    off = 0
    for (h, w), s in zip(_LEVELS, _STRIDES):
        sx = np.arange(w, dtype=np.float32) * s + s // 2
        sy = np.arange(h, dtype=np.float32) * s + s // 2
        ys, xs = np.meshgrid(sy, sx, indexing="ij")
        cx[0, off:off + h * w] = xs.reshape(-1)
        cy[0, off:off + h * w] = ys.reshape(-1)
        off += h * w
    return cx, cy

_CX, _CY = _make_coords()
_LVL_OFF = (0, 4096, 5120, 5376, 5440)


def _score_decode_body(*refs):
    cls_refs = refs[0:5]
    cnt_refs = refs[5:10]
    reg_refs = refs[10:15]
    cx_ref, cy_ref = refs[15], refs[16]
    s_ref, c_ref, x1_ref, y1_ref, x2_ref, y2_ref, t_ref = refs[17:24]

    per_level = []
    for lvl in range(5):
        hw = _LEVELS[lvl][0] * _LEVELS[lvl][1]
        off = _LVL_OFF[lvl]
        logits = cls_refs[lvl][...]
        sig = jax.nn.sigmoid(logits)
        m = jnp.max(sig, axis=1)
        cidx = lax.broadcasted_iota(jnp.int32, (_B, _NC, hw), 1)
        am = jnp.min(jnp.where(sig == m[:, None, :], cidx, 10**9), axis=1)
        cls_out = (am + 1).astype(jnp.float32)
        cnt_sig = jax.nn.sigmoid(cnt_refs[lvl][...])[:, 0, :]
        score = jnp.sqrt(m * cnt_sig)
        reg = reg_refs[lvl][...]
        cx = cx_ref[:, off:off + hw]
        cy = cy_ref[:, off:off + hw]
        x1 = cx - reg[:, 0, :]
        y1 = cy - reg[:, 1, :]
        x2 = cx + reg[:, 2, :]
        y2 = cy + reg[:, 3, :]
        per_level.append((score, cls_out, x1, y1, x2, y2))

    outs = (s_ref, c_ref, x1_ref, y1_ref, x2_ref, y2_ref)
    pads = (-1.0, 0.0, 0.0, 0.0, 0.0, 0.0)
    for lvl in range(3):
        hw = _LEVELS[lvl][0] * _LEVELS[lvl][1]
        off = _LVL_OFF[lvl]
        for ref, val in zip(outs, per_level[lvl]):
            ref[:, off:off + hw] = val
    for j, (ref, padv) in enumerate(zip(outs, pads)):
        tile = jnp.concatenate(
            [per_level[3][j], per_level[4][j],
             jnp.full((_B, 48), padv, jnp.float32)], axis=1)
        ref[:, _LVL_OFF[3]:_LVL_OFF[3] + 128] = tile

    sb = lax.bitcast_convert_type(s_ref[...], jnp.int32)
    lane = lax.broadcasted_iota(jnp.int32, (_B, _NPAD), 1)
    real = lane < _N

    def bs_body(_, carry):
        lo, hi = carry
        mid = lo + lax.shift_right_logical(hi - lo, 1)
        cnt = jnp.sum(jnp.where((sb >= mid) & real, 1, 0), axis=1,
                      keepdims=True)
        ge = cnt >= _K
        return jnp.where(ge, mid, lo), jnp.where(ge, hi, mid)

    lo0 = jnp.zeros((_B, 1), jnp.int32)
    hi0 = jnp.full((_B, 1), 0x7F800000, jnp.int32)
    lo, hi = lax.fori_loop(0, 31, bs_body, (lo0, hi0))
    t_ref[...] = jnp.broadcast_to(lo, (_B, 16))


def _score_decode(cls_list, cnt_list, reg_list):
    out_sh = [jax.ShapeDtypeStruct((_B, _NPAD), jnp.float32)] * 6 + \
             [jax.ShapeDtypeStruct((_B, 16), jnp.int32)]
    fn = pl.pallas_call(_score_decode_body, out_shape=out_sh)
    return fn(*cls_list, *cnt_list, *reg_list, jnp.asarray(_CX), jnp.asarray(_CY))


_NSTEP = _NPAD // 16


def _sc_compact_body(s_hbm, c_hbm, x1_hbm, y1_hbm, x2_hbm, y2_hbm, t_hbm,
                     so_hbm, co_hbm, xo1_hbm, yo1_hbm, xo2_hbm, yo2_hbm,
                     sv, cv, x1v, y1v, x2v, y2v, tv, idxa,
                     os_, oc_, ox1, oy1, ox2, oy2):
    wid = lax.axis_index("s") * 2 + lax.axis_index("c")

    @pl.when(wid < _B)
    def _():
        b = wid
        pltpu.sync_copy(t_hbm.at[b], tv)
        pltpu.sync_copy(s_hbm.at[b], sv)
        pltpu.sync_copy(c_hbm.at[b], cv)
        pltpu.sync_copy(x1_hbm.at[b], x1v)
        pltpu.sync_copy(y1_hbm.at[b], y1v)
        pltpu.sync_copy(x2_hbm.at[b], x2v)
        pltpu.sync_copy(y2_hbm.at[b], y2v)

        t = tv[...]
        lane16 = lax.iota(jnp.int32, 16)
        sent = jnp.full((16,), _NPAD - 1, jnp.int32)

        @pl.loop(0, _KPAD, step=16)
        def _fill(j):
            idxa[pl.ds(pl.multiple_of(j, 16), 16)] = sent

        def make_pass(cmp_eq):
            def body(i, base):
                off = pl.multiple_of(i * 16, 16)
                bits = plsc.bitcast(sv[pl.ds(off, 16)], jnp.int32)
                m = (bits == t) if cmp_eq else (bits > t)
                mi = m.astype(jnp.int32)
                posv = base + plsc.cumsum(mi) - mi
                plsc.store_scatter(idxa, [posv], lane16 + i * 16, mask=m)
                return base + plsc.all_reduce_population_count(m)
            return body

        gtot = lax.fori_loop(0, _NSTEP, make_pass(False),
                             jnp.zeros((16,), jnp.int32))
        lax.fori_loop(0, _NSTEP, make_pass(True), gtot)

        @pl.loop(0, _KPAD, step=16)
        def _gather(j):
            jj = pl.multiple_of(j, 16)
            idxv = idxa[pl.ds(jj, 16)]
            os_[pl.ds(jj, 16)] = plsc.load_gather(sv, [idxv])
            oc_[pl.ds(jj, 16)] = plsc.load_gather(cv, [idxv])
            ox1[pl.ds(jj, 16)] = plsc.load_gather(x1v, [idxv])
            oy1[pl.ds(jj, 16)] = plsc.load_gather(y1v, [idxv])
            ox2[pl.ds(jj, 16)] = plsc.load_gather(x2v, [idxv])
            oy2[pl.ds(jj, 16)] = plsc.load_gather(y2v, [idxv])

        pltpu.sync_copy(os_, so_hbm.at[b])
        pltpu.sync_copy(oc_, co_hbm.at[b])
        pltpu.sync_copy(ox1, xo1_hbm.at[b])
        pltpu.sync_copy(oy1, yo1_hbm.at[b])
        pltpu.sync_copy(ox2, xo2_hbm.at[b])
        pltpu.sync_copy(oy2, yo2_hbm.at[b])


def _sc_compiler_params():
    cp = pltpu.CompilerParams()
    if "needs_layout_passes" in pltpu.CompilerParams.__dataclass_fields__:
        cp = dataclasses.replace(cp, needs_layout_passes=False)
    return cp


def _compact(s, c, x1, y1, x2, y2, t):
    mesh = plsc.VectorSubcoreMesh(core_axis_name="c", subcore_axis_name="s")
    fn = pl.kernel(
        _sc_compact_body,
        out_type=[jax.ShapeDtypeStruct((_B, _KPAD), jnp.float32)] * 6,
        mesh=mesh,
        compiler_params=_sc_compiler_params(),
        scratch_types=[pltpu.VMEM((_NPAD,), jnp.float32)] * 6
        + [pltpu.VMEM((16,), jnp.int32),
           pltpu.VMEM((_N + 16,), jnp.int32)]
        + [pltpu.VMEM((_KPAD,), jnp.float32)] * 6,
    )
    return fn(s, c, x1, y1, x2, y2, t)


_T = 128
_NBLK = _KPAD // _T


def _nms_body(sc_ref, cl_ref, x1_ref, y1_ref, x2_ref, y2_ref,
              so_ref, co_ref, bx1_ref, by1_ref, bx2_ref, by2_ref,
              x1s, y1s, x2s, y2s, ar, kp, x1sT, y1sT, x2sT, y2sT, arT, kpT):
    sb = lax.bitcast_convert_type(sc_ref[...], jnp.int32)
    sbT = jnp.transpose(sb)
    qi = lax.broadcasted_iota(jnp.int32, (_KPAD, _KPAD), 1)
    pi = lax.broadcasted_iota(jnp.int32, (_KPAD, _KPAD), 0)
    r_row = lax.broadcasted_iota(jnp.int32, (_KPAD, _KPAD), 1)
    chans = [sc_ref[...], cl_ref[...], x1_ref[...], y1_ref[...],
             x2_ref[...], y2_ref[...]]
    sorted_chans = []
    for b in range(_B):
        colb = sbT[:, b:b + 1]
        rowb = sb[b:b + 1, :]
        before = (rowb > colb) | ((rowb == colb) & (qi < pi))
        rkp = jnp.sum(before.astype(jnp.float32), axis=1,
                      keepdims=True).astype(jnp.int32)
        oh = jnp.where(rkp == r_row, 1.0, 0.0).astype(jnp.bfloat16)
        vb = jnp.concatenate([ch[b:b + 1, :] for ch in chans], axis=0)
        hi = vb.astype(jnp.bfloat16)
        r1 = vb - hi.astype(jnp.float32)
        md = r1.astype(jnp.bfloat16)
        lo2 = (r1 - md.astype(jnp.float32)).astype(jnp.bfloat16)
        split = jnp.concatenate([hi, md, lo2], axis=0)
        acc3 = lax.dot_general(split, oh, (((1,), (0,)), ((), ())),
                               preferred_element_type=jnp.float32)
        sorted_chans.append(acc3[0:6] + acc3[6:12] + acc3[12:18])

    scores = jnp.concatenate([sc[0:1, :] for sc in sorted_chans], axis=0)
    classes = jnp.concatenate([sc[1:2, :] for sc in sorted_chans], axis=0)
    x1 = jnp.concatenate([sc[2:3, :] for sc in sorted_chans], axis=0)
    y1 = jnp.concatenate([sc[3:4, :] for sc in sorted_chans], axis=0)
    x2 = jnp.concatenate([sc[4:5, :] for sc in sorted_chans], axis=0)
    y2 = jnp.concatenate([sc[5:6, :] for sc in sorted_chans], axis=0)

    lane = lax.broadcasted_iota(jnp.int32, (_B, _KPAD), 1)
    live = lane < _K
    neg = jnp.float32(-3.0e38)
    mc = jnp.max(
        jnp.maximum(jnp.maximum(jnp.where(live, x1, neg), jnp.where(live, y1, neg)),
                    jnp.maximum(jnp.where(live, x2, neg), jnp.where(live, y2, neg))),
        axis=1, keepdims=True)
    mcp1 = mc + 1.0
    offs = classes * mcp1
    x1s[...] = x1 + offs
    y1s[...] = y1 + offs
    x2s[...] = x2 + offs
    y2s[...] = y2 + offs
    ar[...] = (x2s[...] - x1s[...] + 1.0) * (y2s[...] - y1s[...] + 1.0)
    kp[...] = (scores >= _SCORE_THR).astype(jnp.float32)
    x1sT[...] = jnp.transpose(x1s[...])
    y1sT[...] = jnp.transpose(y1s[...])
    x2sT[...] = jnp.transpose(x2s[...])
    y2sT[...] = jnp.transpose(y2s[...])
    arT[...] = jnp.transpose(ar[...])

    lane_t = lax.broadcasted_iota(jnp.int32, (_B, _T), 1)
    for blk in range(_NBLK):
        base = blk * _T
        nb = min(_T, _K - base)
        bx1 = x1s[:, base:base + _T]
        by1 = y1s[:, base:base + _T]
        bx2 = x2s[:, base:base + _T]
        by2 = y2s[:, base:base + _T]
        bar = ar[:, base:base + _T]

        def body(i, kb):
            oh = jnp.where(lane_t == i, 1.0, 0.0)
            ext = lambda a: jnp.sum(a * oh, axis=1, keepdims=True)
            cx1, cy1, cx2, cy2, car = map(ext, (bx1, by1, bx2, by2, bar))
            ck = jnp.sum(kb * oh, axis=1, keepdims=True)
            xx1 = jnp.maximum(cx1, bx1)
            yy1 = jnp.maximum(cy1, by1)
            xx2 = jnp.minimum(cx2, bx2)
            yy2 = jnp.minimum(cy2, by2)
            inter = jnp.maximum(xx2 - xx1, 0.0) * jnp.maximum(yy2 - yy1, 0.0)
            iou = inter / (car + bar - inter)
            sup = (iou > _NMS_IOU) & (lane_t > i) & (ck > 0.0)
            return jnp.where(sup, 0.0, kb)

        kb = lax.fori_loop(0, nb, body, kp[:, base:base + _T])
        kp[:, base:base + _T] = kb

        if blk < _NBLK - 1:
            rest = _KPAD - base - _T
            kbT = jnp.transpose(kb)
            for b in range(_B):
                cols = lambda ref: ref[base:base + _T, b:b + 1]
                rx1 = x1s[b:b + 1, base + _T:]
                ry1 = y1s[b:b + 1, base + _T:]
                rx2 = x2s[b:b + 1, base + _T:]
                ry2 = y2s[b:b + 1, base + _T:]
                rar = ar[b:b + 1, base + _T:]
                xx1 = jnp.maximum(cols(x1sT), rx1)
                yy1 = jnp.maximum(cols(y1sT), ry1)
                xx2 = jnp.minimum(cols(x2sT), rx2)
                yy2 = jnp.minimum(cols(y2sT), ry2)
                inter = jnp.maximum(xx2 - xx1, 0.0) * jnp.maximum(yy2 - yy1, 0.0)
                iou = inter / (cols(arT) + rar - inter)
                supm = (iou > _NMS_IOU) & (kbT[:, b:b + 1] > 0.0)
                supj = jnp.max(supm.astype(jnp.float32), axis=0, keepdims=True)
                kp[b:b + 1, base + _T:] = jnp.where(
                    supj > 0.0, 0.0, kp[b:b + 1, base + _T:])

    kf = kp[...]
    so_ref[...] = scores * kf
    co_ref[...] = (classes * kf).astype(jnp.int32)
    bx1_ref[...] = x1 * kf
    by1_ref[...] = y1 * kf
    bx2_ref[...] = x2 * kf
    by2_ref[...] = y2 * kf


def _nms(topv, topc, tx1, ty1, tx2, ty2):
    out_sh = [jax.ShapeDtypeStruct((_B, _KPAD), jnp.float32),
              jax.ShapeDtypeStruct((_B, _KPAD), jnp.int32)] + \
             [jax.ShapeDtypeStruct((_B, _KPAD), jnp.float32)] * 4
    fn = pl.pallas_call(
        _nms_body,
        out_shape=out_sh,
        scratch_shapes=[pltpu.VMEM((_B, _KPAD), jnp.float32)] * 6
        + [pltpu.VMEM((_KPAD, _B), jnp.float32)] * 6,
    )
    return fn(topv, topc, tx1, ty1, tx2, ty2)


def kernel(cls_p3, cls_p4, cls_p5, cls_p6, cls_p7,
           cnt_p3, cnt_p4, cnt_p5, cnt_p6, cnt_p7,
           reg_p3, reg_p4, reg_p5, reg_p6, reg_p7):
    cls_list = [c.reshape(_B, _NC, -1)
                for c in (cls_p3, cls_p4, cls_p5, cls_p6, cls_p7)]
    cnt_list = [c.reshape(_B, 1, -1)
                for c in (cnt_p3, cnt_p4, cnt_p5, cnt_p6, cnt_p7)]
    reg_list = [r.reshape(_B, 4, -1)
                for r in (reg_p3, reg_p4, reg_p5, reg_p6, reg_p7)]

    scores, classes, x1, y1, x2, y2, t = _score_decode(cls_list, cnt_list,
                                                       reg_list)
    cs, cc, cx1, cy1, cx2, cy2 = _compact(scores, classes, x1, y1, x2, y2, t)
    so, co, bx1, by1, bx2, by2 = _nms(cs, cc, cx1, cy1, cx2, cy2)

    boxes = jnp.stack([bx1[:, :_K], by1[:, :_K], bx2[:, :_K], by2[:, :_K]],
                      axis=-1)
    return so[:, :_K], co[:, :_K], boxes
